# baseline (speedup 1.0000x reference)
import functools

import jax
import jax.numpy as jnp
from jax import lax
from jax.experimental import pallas as pl
from jax.experimental.pallas import tpu as pltpu
from jax.experimental.pallas import tpu_sc as plsc

_NUM_CORES = 2
_NUM_SUBCORES = 16
_NUM_WORKERS = _NUM_CORES * _NUM_SUBCORES


def _pool_chunk(gbuf, pooled_v, row0, rows, L):
  inv = jnp.float32(1.0 / L)

  def body(r, _):
    acc = [jnp.zeros((16,), jnp.float32) for _ in range(4)]
    for j in range(L):
      acc[j % 4] = acc[j % 4] + gbuf[r, j]
    pooled_v[row0 + r] = ((acc[0] + acc[1]) + (acc[2] + acc[3])) * inv
    return 0

  lax.fori_loop(0, rows, body, 0)


def _make_gather_pool(B, V, D, L, chunk_rows):
  assert D == 16
  rows_per_w = B // _NUM_WORKERS
  assert rows_per_w % (2 * chunk_rows) == 0
  n_chunks = rows_per_w // chunk_rows
  n_pairs = n_chunks // 2

  mesh = plsc.VectorSubcoreMesh(
      core_axis_name="c", subcore_axis_name="s",
      num_cores=_NUM_CORES, num_subcores=_NUM_SUBCORES)

  @functools.partial(
      pl.kernel,
      out_type=jax.ShapeDtypeStruct((B, D), jnp.float32),
      mesh=mesh,
      scratch_types=[
          [pltpu.VMEM((chunk_rows, L), jnp.int32)] * 2,
          [pltpu.VMEM((chunk_rows, L, D), jnp.float32)] * 2,
          pltpu.VMEM((rows_per_w, D), jnp.float32),
          [pltpu.SemaphoreType.DMA] * 2,
      ],
      compiler_params=pltpu.CompilerParams(use_tc_tiling_on_sc=False),
  )
  def gather_pool(table_hbm, idx_hbm, out_hbm, idx_v, gbuf, pooled_v, sems):
    wid = lax.axis_index("s") * _NUM_CORES + lax.axis_index("c")
    row_base = wid * rows_per_w

    def copy_idx(c, b):
      pltpu.sync_copy(
          idx_hbm.at[pl.ds(row_base + c * chunk_rows, chunk_rows)], idx_v[b])

    def fire(b):
      def go(r, _):
        pltpu.async_copy(
            table_hbm.at[idx_v[b].at[r]], gbuf[b].at[r], sems[b])
        return 0
      lax.fori_loop(0, chunk_rows, go, 0)

    def drain(b):
      def dr(r, _):
        pltpu.make_async_copy(
            table_hbm.at[idx_v[b].at[r]], gbuf[b].at[r], sems[b]).wait()
        return 0
      lax.fori_loop(0, chunk_rows, dr, 0)

    for b in (0, 1):
      copy_idx(b, b)
      fire(b)

    def pair_body(p, _):
      for b in (0, 1):
        c = 2 * p + b
        drain(b)
        @pl.when(c < n_chunks - 2)
        def _():
          copy_idx(c + 2, b)
        _pool_chunk(gbuf[b], pooled_v, c * chunk_rows, chunk_rows, L)

        @pl.when(c < n_chunks - 2)
        def _():
          fire(b)
      return 0

    lax.fori_loop(0, n_pairs, pair_body, 0)
    pltpu.sync_copy(pooled_v, out_hbm.at[pl.ds(row_base, rows_per_w)])

  return gather_pool


_TCOLS = 1024


def _make_transpose(V, D):
  assert D == 16
  units = V // _TCOLS
  rem = V % _TCOLS
  assert rem % 8 == 0

  mesh = plsc.VectorSubcoreMesh(
      core_axis_name="c", subcore_axis_name="s",
      num_cores=_NUM_CORES, num_subcores=_NUM_SUBCORES)

  base_cnt = units // _NUM_WORKERS
  extra = units % _NUM_WORKERS

  @functools.partial(
      pl.kernel,
      out_type=jax.ShapeDtypeStruct((V * D,), jnp.float32),
      mesh=mesh,
      scratch_types=[
          [pltpu.VMEM((D, _TCOLS), jnp.float32)] * 2,
          [pltpu.VMEM((_TCOLS * D,), jnp.float32)] * 2,
          pltpu.VMEM((D, rem), jnp.float32) if rem else None,
          [pltpu.SemaphoreType.DMA] * 2,
          [pltpu.SemaphoreType.DMA] * 2,
      ],
      compiler_params=pltpu.CompilerParams(
          use_tc_tiling_on_sc=True, needs_layout_passes=False),
  )
  def transpose(tin_hbm, out_hbm, slab_v, obuf_v, tail_v, isems, osems):
    wid = lax.axis_index("s") * _NUM_CORES + lax.axis_index("c")
    iotas = [lax.iota(jnp.int32, 16) * D + f for f in range(D)]
    cnt = base_cnt + jnp.where(wid < extra, 1, 0)
    u0 = wid * base_cnt + jnp.minimum(wid, extra)

    def in_cp(u, b):
      return pltpu.make_async_copy(
          tin_hbm.at[:, pl.ds(u * _TCOLS, _TCOLS)], slab_v[b], isems[b])

    def out_cp(u, b):
      return pltpu.make_async_copy(
          obuf_v[b], out_hbm.at[pl.ds(u * _TCOLS * D, _TCOLS * D)], osems[b])

    def scatter(slab, obuf, ncols):
      for s in range(ncols // 16):
        seg = obuf.at[pl.ds(s * 16 * D, 16 * D)]
        for f in range(D):
          plsc.store_scatter(seg, [iotas[f]], slab[f, pl.ds(16 * s, 16)])

    in_cp(u0, 0).start()
    in_cp(u0 + 1, 1).start()

    def pair_body(p, _):
      for b in (0, 1):
        rel = 2 * p + b
        u = u0 + rel

        @pl.when(rel < cnt)
        def _():
          in_cp(u, b).wait()

          @pl.when(rel + 2 < cnt)
          def _():
            in_cp(u + 2, b).start()

          @pl.when(rel >= 2)
          def _():
            out_cp(u, b).wait()
          scatter(slab_v[b], obuf_v[b], _TCOLS)
          out_cp(u, b).start()
      return 0

    lax.fori_loop(0, (base_cnt + 2) // 2, pair_body, 0)
    for b in (0, 1):
      pltpu.make_async_copy(
          obuf_v[b], out_hbm.at[pl.ds(0, _TCOLS * D)], osems[b]).wait()

    if rem:
      @pl.when(wid == 0)
      def _():
        col0 = units * _TCOLS
        pltpu.sync_copy(tin_hbm.at[:, pl.ds(col0, rem)], tail_v)
        scatter(tail_v, obuf_v[0], rem)
        pltpu.sync_copy(obuf_v[0].at[pl.ds(0, rem * D)],
                        out_hbm.at[pl.ds(col0 * D, rem * D)])

  return transpose


def _mlp_body(ts, sg, hr, dy, mo, dw, wk, fc, pooled, W1, b1, W2, b2, W3, b3,
              out):
  h = pooled[...] @ W1[8:24, :] + b1[0, :]
  scalars = (ts, sg, hr, dy, mo, dw, wk, fc)
  for i, s in enumerate(scalars):
    h = h + s[...].reshape(-1, 1) * W1[i, :]
  h = jnp.where(h >= 0, h, 0.001 * h)
  h = h @ W2[...] + b2[0, :]
  h = h @ W3[...] + b3[0, :]
  out[...] = jax.nn.sigmoid(h[:, 0])


def _make_mlp(B, D, blk):
  grid = B // blk
  scal_spec = pl.BlockSpec((blk,), lambda i: (i,))
  full = lambda shape: pl.BlockSpec(shape, lambda i: (0,) * len(shape))
  return pl.pallas_call(
      _mlp_body,
      grid=(grid,),
      in_specs=[scal_spec] * 8 + [
          pl.BlockSpec((blk, D), lambda i: (i, 0)),
          full((24, 64)), full((1, 64)),
          full((64, 32)), full((1, 32)),
          full((32, 1)), full((1, 1)),
      ],
      out_specs=pl.BlockSpec((blk,), lambda i: (i,)),
      out_shape=jax.ShapeDtypeStruct((B,), jnp.float32),
  )


@jax.jit
def kernel(time_step, sign, hour, day, month, day_of_week, is_weekend,
           fbas_count, fbas_indices, emb_table, W1, b1, W2, b2, W3, b3):
  B, L = fbas_indices.shape
  V, D = emb_table.shape

  table_rm = _make_transpose(V, D)(emb_table.T).reshape(V, D)
  pooled = _make_gather_pool(B, V, D, L, chunk_rows=32)(
      table_rm, fbas_indices)

  out = _make_mlp(B, D, blk=4096)(
      time_step, sign, hour, day, month, day_of_week, is_weekend,
      fbas_count, pooled,
      W1, b1.reshape(1, 64), W2, b2.reshape(1, 32), W3, b3.reshape(1, 1))
  return out.reshape(B, 1)


# speedup vs baseline: 1.0376x; 1.0376x over previous
import functools

import jax
import jax.numpy as jnp
from jax import lax
from jax.experimental import pallas as pl
from jax.experimental.pallas import tpu as pltpu
from jax.experimental.pallas import tpu_sc as plsc

_NUM_CORES = 2
_NUM_SUBCORES = 16
_NUM_WORKERS = _NUM_CORES * _NUM_SUBCORES


def _pool_chunk(gbuf, pooled_v, row0, rows, L):
  inv = jnp.float32(1.0 / L)

  def body(r, _):
    acc = [jnp.zeros((16,), jnp.float32) for _ in range(4)]
    for j in range(L):
      acc[j % 4] = acc[j % 4] + gbuf[r, j]
    pooled_v[row0 + r] = ((acc[0] + acc[1]) + (acc[2] + acc[3])) * inv
    return 0

  lax.fori_loop(0, rows, body, 0)


def _make_gather_pool(B, V, D, L, chunk_rows):
  assert D == 16
  rows_per_w = B // _NUM_WORKERS
  assert rows_per_w % (2 * chunk_rows) == 0
  n_chunks = rows_per_w // chunk_rows
  n_pairs = n_chunks // 2

  mesh = plsc.VectorSubcoreMesh(
      core_axis_name="c", subcore_axis_name="s",
      num_cores=_NUM_CORES, num_subcores=_NUM_SUBCORES)

  @functools.partial(
      pl.kernel,
      out_type=jax.ShapeDtypeStruct((B, D), jnp.float32),
      mesh=mesh,
      scratch_types=[
          [pltpu.VMEM((chunk_rows, L), jnp.int32)] * 2,
          [pltpu.VMEM((chunk_rows, L, D), jnp.float32)] * 2,
          pltpu.VMEM((rows_per_w, D), jnp.float32),
          [pltpu.SemaphoreType.DMA] * 2,
      ],
      compiler_params=pltpu.CompilerParams(use_tc_tiling_on_sc=False),
  )
  def gather_pool(table_hbm, idx_hbm, out_hbm, idx_v, gbuf, pooled_v, sems):
    wid = lax.axis_index("s") * _NUM_CORES + lax.axis_index("c")
    row_base = wid * rows_per_w

    def copy_idx(c, b):
      pltpu.sync_copy(
          idx_hbm.at[pl.ds(row_base + c * chunk_rows, chunk_rows)], idx_v[b])

    def fire(b):
      def go(r, _):
        pltpu.async_copy(
            table_hbm.at[idx_v[b].at[r]], gbuf[b].at[r], sems[b])
        return 0
      lax.fori_loop(0, chunk_rows, go, 0)

    def drain(b):
      def dr(r, _):
        pltpu.make_async_copy(
            table_hbm.at[idx_v[b].at[r]], gbuf[b].at[r], sems[b]).wait()
        return 0
      lax.fori_loop(0, chunk_rows, dr, 0)

    for b in (0, 1):
      copy_idx(b, b)
      fire(b)

    def pair_body(p, _):
      for b in (0, 1):
        c = 2 * p + b
        drain(b)
        @pl.when(c < n_chunks - 2)
        def _():
          copy_idx(c + 2, b)
        _pool_chunk(gbuf[b], pooled_v, c * chunk_rows, chunk_rows, L)

        @pl.when(c < n_chunks - 2)
        def _():
          fire(b)
      return 0

    lax.fori_loop(0, n_pairs, pair_body, 0)
    pltpu.sync_copy(pooled_v, out_hbm.at[pl.ds(row_base, rows_per_w)])

  return gather_pool


_TCOLS = 512


def _make_transpose(V, D):
  assert D == 16
  units = V // _TCOLS
  rem = V % _TCOLS
  assert rem % 8 == 0

  mesh = plsc.VectorSubcoreMesh(
      core_axis_name="c", subcore_axis_name="s",
      num_cores=_NUM_CORES, num_subcores=_NUM_SUBCORES)

  base_cnt = units // _NUM_WORKERS
  extra = units % _NUM_WORKERS

  @functools.partial(
      pl.kernel,
      out_type=jax.ShapeDtypeStruct((V * D,), jnp.float32),
      mesh=mesh,
      scratch_types=[
          [pltpu.VMEM((D, _TCOLS), jnp.float32)] * 2,
          [pltpu.VMEM((_TCOLS * D,), jnp.float32)] * 2,
          pltpu.VMEM((D, rem), jnp.float32) if rem else None,
          [pltpu.SemaphoreType.DMA] * 2,
          [pltpu.SemaphoreType.DMA] * 2,
      ],
      compiler_params=pltpu.CompilerParams(
          use_tc_tiling_on_sc=True, needs_layout_passes=False),
  )
  def transpose(tin_hbm, out_hbm, slab_v, obuf_v, tail_v, isems, osems):
    wid = lax.axis_index("s") * _NUM_CORES + lax.axis_index("c")
    iotas = [lax.iota(jnp.int32, 16) * D + f for f in range(D)]
    cnt = base_cnt + jnp.where(wid < extra, 1, 0)
    u0 = wid * base_cnt + jnp.minimum(wid, extra)

    def in_cp(u, b):
      return pltpu.make_async_copy(
          tin_hbm.at[:, pl.ds(u * _TCOLS, _TCOLS)], slab_v[b], isems[b])

    def out_cp(u, b):
      return pltpu.make_async_copy(
          obuf_v[b], out_hbm.at[pl.ds(u * _TCOLS * D, _TCOLS * D)], osems[b])

    def scatter(slab, obuf, ncols):
      for s in range(ncols // 16):
        seg = obuf.at[pl.ds(s * 16 * D, 16 * D)]
        for f in range(D):
          plsc.store_scatter(seg, [iotas[f]], slab[f, pl.ds(16 * s, 16)])

    in_cp(u0, 0).start()
    in_cp(u0 + 1, 1).start()

    def pair_body(p, _):
      for b in (0, 1):
        rel = 2 * p + b
        u = u0 + rel

        @pl.when(rel < cnt)
        def _():
          in_cp(u, b).wait()

          @pl.when(rel + 2 < cnt)
          def _():
            in_cp(u + 2, b).start()

          @pl.when(rel >= 2)
          def _():
            out_cp(u, b).wait()
          scatter(slab_v[b], obuf_v[b], _TCOLS)
          out_cp(u, b).start()
      return 0

    lax.fori_loop(0, (base_cnt + 2) // 2, pair_body, 0)
    for b in (0, 1):
      pltpu.make_async_copy(
          obuf_v[b], out_hbm.at[pl.ds(0, _TCOLS * D)], osems[b]).wait()

    if rem:
      @pl.when(wid == 0)
      def _():
        col0 = units * _TCOLS
        pltpu.sync_copy(tin_hbm.at[:, pl.ds(col0, rem)], tail_v)
        scatter(tail_v, obuf_v[0], rem)
        pltpu.sync_copy(obuf_v[0].at[pl.ds(0, rem * D)],
                        out_hbm.at[pl.ds(col0 * D, rem * D)])

  return transpose


def _mlp_body(ts, sg, hr, dy, mo, dw, wk, fc, pooled, W1, b1, W2, b2, W3, b3,
              out):
  h = pooled[...] @ W1[8:24, :] + b1[0, :]
  scalars = (ts, sg, hr, dy, mo, dw, wk, fc)
  for i, s in enumerate(scalars):
    h = h + s[...].reshape(-1, 1) * W1[i, :]
  h = jnp.where(h >= 0, h, 0.001 * h)
  h = h @ W2[...] + b2[0, :]
  h = h @ W3[...] + b3[0, :]
  out[...] = jax.nn.sigmoid(h[:, 0])


def _make_mlp(B, D, blk):
  grid = B // blk
  scal_spec = pl.BlockSpec((blk,), lambda i: (i,))
  full = lambda shape: pl.BlockSpec(shape, lambda i: (0,) * len(shape))
  return pl.pallas_call(
      _mlp_body,
      grid=(grid,),
      in_specs=[scal_spec] * 8 + [
          pl.BlockSpec((blk, D), lambda i: (i, 0)),
          full((24, 64)), full((1, 64)),
          full((64, 32)), full((1, 32)),
          full((32, 1)), full((1, 1)),
      ],
      out_specs=pl.BlockSpec((blk,), lambda i: (i,)),
      out_shape=jax.ShapeDtypeStruct((B,), jnp.float32),
  )


@jax.jit
def kernel(time_step, sign, hour, day, month, day_of_week, is_weekend,
           fbas_count, fbas_indices, emb_table, W1, b1, W2, b2, W3, b3):
  B, L = fbas_indices.shape
  V, D = emb_table.shape

  table_rm = _make_transpose(V, D)(emb_table.T).reshape(V, D)
  pooled = _make_gather_pool(B, V, D, L, chunk_rows=32)(
      table_rm, fbas_indices)

  out = _make_mlp(B, D, blk=4096)(
      time_step, sign, hour, day, month, day_of_week, is_weekend,
      fbas_count, pooled,
      W1, b1.reshape(1, 64), W2, b2.reshape(1, 32), W3, b3.reshape(1, 1))
  return out.reshape(B, 1)


# speedup vs baseline: 1.1209x; 1.0803x over previous
import functools

import jax
import jax.numpy as jnp
from jax import lax
from jax.experimental import pallas as pl
from jax.experimental.pallas import tpu as pltpu
from jax.experimental.pallas import tpu_sc as plsc

_NUM_CORES = 2
_NUM_SUBCORES = 16
_NUM_WORKERS = _NUM_CORES * _NUM_SUBCORES


def _pool_chunk(gbuf, pooled_v, row0, rows, L):
  inv = jnp.float32(1.0 / L)

  def body(r, _):
    acc = [jnp.zeros((16,), jnp.float32) for _ in range(4)]
    for j in range(L):
      acc[j % 4] = acc[j % 4] + gbuf[r * L + j]
    pooled_v[row0 + r] = ((acc[0] + acc[1]) + (acc[2] + acc[3])) * inv
    return 0

  lax.fori_loop(0, rows, body, 0)


_NBUF = 4


def _make_gather_pool(B, V, D, L, chunk_rows):
  assert D == 16
  rows_per_w = B // _NUM_WORKERS
  n_chunks = rows_per_w // chunk_rows
  assert n_chunks % _NBUF == 0
  chunk_idx = chunk_rows * L

  mesh = plsc.VectorSubcoreMesh(
      core_axis_name="c", subcore_axis_name="s",
      num_cores=_NUM_CORES, num_subcores=_NUM_SUBCORES)

  @functools.partial(
      pl.kernel,
      out_type=jax.ShapeDtypeStruct((B, D), jnp.float32),
      mesh=mesh,
      scratch_types=[
          pltpu.VMEM((rows_per_w * L,), jnp.int32),
          [pltpu.VMEM((chunk_idx, D), jnp.float32)] * _NBUF,
          pltpu.VMEM((rows_per_w, D), jnp.float32),
          [pltpu.SemaphoreType.DMA] * _NBUF,
      ],
      compiler_params=pltpu.CompilerParams(use_tc_tiling_on_sc=False),
  )
  def gather_pool(table_hbm, idx_hbm, out_hbm, idx_v, gbuf, pooled_v, sems):
    wid = lax.axis_index("s") * _NUM_CORES + lax.axis_index("c")

    def cp(c, b):
      return pltpu.make_async_copy(
          table_hbm.at[idx_v.at[pl.ds(c * chunk_idx, chunk_idx)]],
          gbuf[b], sems[b])

    pltpu.sync_copy(idx_hbm.at[wid], idx_v)
    for b in range(_NBUF):
      cp(b, b).start()

    def quad_body(p, _):
      for b in range(_NBUF):
        c = p * _NBUF + b
        cp(c, b).wait()
        _pool_chunk(gbuf[b], pooled_v, c * chunk_rows, chunk_rows, L)

        @pl.when(p < n_chunks // _NBUF - 1)
        def _():
          cp(c + _NBUF, b).start()
      return 0

    lax.fori_loop(0, n_chunks // _NBUF, quad_body, 0)
    pltpu.sync_copy(pooled_v, out_hbm.at[pl.ds(wid * rows_per_w, rows_per_w)])

  return gather_pool


_TCOLS = 512


def _make_transpose(V, D):
  assert D == 16
  units = V // _TCOLS
  rem = V % _TCOLS
  assert rem % 8 == 0

  mesh = plsc.VectorSubcoreMesh(
      core_axis_name="c", subcore_axis_name="s",
      num_cores=_NUM_CORES, num_subcores=_NUM_SUBCORES)

  base_cnt = units // _NUM_WORKERS
  extra = units % _NUM_WORKERS

  @functools.partial(
      pl.kernel,
      out_type=jax.ShapeDtypeStruct((V * D,), jnp.float32),
      mesh=mesh,
      scratch_types=[
          [pltpu.VMEM((D, _TCOLS), jnp.float32)] * 2,
          [pltpu.VMEM((_TCOLS * D,), jnp.float32)] * 2,
          pltpu.VMEM((D, rem), jnp.float32) if rem else None,
          [pltpu.SemaphoreType.DMA] * 2,
          [pltpu.SemaphoreType.DMA] * 2,
      ],
      compiler_params=pltpu.CompilerParams(
          use_tc_tiling_on_sc=True, needs_layout_passes=False),
  )
  def transpose(tin_hbm, out_hbm, slab_v, obuf_v, tail_v, isems, osems):
    wid = lax.axis_index("s") * _NUM_CORES + lax.axis_index("c")
    iotas = [lax.iota(jnp.int32, 16) * D + f for f in range(D)]
    cnt = base_cnt + jnp.where(wid < extra, 1, 0)
    u0 = wid * base_cnt + jnp.minimum(wid, extra)

    def in_cp(u, b):
      return pltpu.make_async_copy(
          tin_hbm.at[:, pl.ds(u * _TCOLS, _TCOLS)], slab_v[b], isems[b])

    def out_cp(u, b):
      return pltpu.make_async_copy(
          obuf_v[b], out_hbm.at[pl.ds(u * _TCOLS * D, _TCOLS * D)], osems[b])

    def scatter(slab, obuf, ncols):
      for s in range(ncols // 16):
        seg = obuf.at[pl.ds(s * 16 * D, 16 * D)]
        for f in range(D):
          plsc.store_scatter(seg, [iotas[f]], slab[f, pl.ds(16 * s, 16)])

    in_cp(u0, 0).start()
    in_cp(u0 + 1, 1).start()

    def pair_body(p, _):
      for b in (0, 1):
        rel = 2 * p + b
        u = u0 + rel

        @pl.when(rel < cnt)
        def _():
          in_cp(u, b).wait()

          @pl.when(rel + 2 < cnt)
          def _():
            in_cp(u + 2, b).start()

          @pl.when(rel >= 2)
          def _():
            out_cp(u, b).wait()
          scatter(slab_v[b], obuf_v[b], _TCOLS)
          out_cp(u, b).start()
      return 0

    lax.fori_loop(0, (base_cnt + 2) // 2, pair_body, 0)
    for b in (0, 1):
      pltpu.make_async_copy(
          obuf_v[b], out_hbm.at[pl.ds(0, _TCOLS * D)], osems[b]).wait()

    if rem:
      @pl.when(wid == 0)
      def _():
        col0 = units * _TCOLS
        pltpu.sync_copy(tin_hbm.at[:, pl.ds(col0, rem)], tail_v)
        scatter(tail_v, obuf_v[0], rem)
        pltpu.sync_copy(obuf_v[0].at[pl.ds(0, rem * D)],
                        out_hbm.at[pl.ds(col0 * D, rem * D)])

  return transpose


def _mlp_body(ts, sg, hr, dy, mo, dw, wk, fc, pooled, W1, b1, W2, b2, W3, b3,
              out):
  h = pooled[...] @ W1[8:24, :] + b1[0, :]
  scalars = (ts, sg, hr, dy, mo, dw, wk, fc)
  for i, s in enumerate(scalars):
    h = h + s[...].reshape(-1, 1) * W1[i, :]
  h = jnp.where(h >= 0, h, 0.001 * h)
  h = h @ W2[...] + b2[0, :]
  h = h @ W3[...] + b3[0, :]
  out[...] = jax.nn.sigmoid(h[:, 0])


def _make_mlp(B, D, blk):
  grid = B // blk
  scal_spec = pl.BlockSpec((blk,), lambda i: (i,))
  full = lambda shape: pl.BlockSpec(shape, lambda i: (0,) * len(shape))
  return pl.pallas_call(
      _mlp_body,
      grid=(grid,),
      in_specs=[scal_spec] * 8 + [
          pl.BlockSpec((blk, D), lambda i: (i, 0)),
          full((24, 64)), full((1, 64)),
          full((64, 32)), full((1, 32)),
          full((32, 1)), full((1, 1)),
      ],
      out_specs=pl.BlockSpec((blk,), lambda i: (i,)),
      out_shape=jax.ShapeDtypeStruct((B,), jnp.float32),
  )


@jax.jit
def kernel(time_step, sign, hour, day, month, day_of_week, is_weekend,
           fbas_count, fbas_indices, emb_table, W1, b1, W2, b2, W3, b3):
  B, L = fbas_indices.shape
  V, D = emb_table.shape

  table_rm = _make_transpose(V, D)(emb_table.T).reshape(V, D)
  idx_by_worker = fbas_indices.reshape(_NUM_WORKERS, (B // _NUM_WORKERS) * L)
  pooled = _make_gather_pool(B, V, D, L, chunk_rows=16)(
      table_rm, idx_by_worker)

  out = _make_mlp(B, D, blk=4096)(
      time_step, sign, hour, day, month, day_of_week, is_weekend,
      fbas_count, pooled,
      W1, b1.reshape(1, 64), W2, b2.reshape(1, 32), W3, b3.reshape(1, 1))
  return out.reshape(B, 1)
